# baseline (speedup 1.0000x reference)
import functools

import jax
import jax.numpy as jnp
from jax import lax
from jax.experimental import pallas as pl
from jax.experimental.pallas import tpu as pltpu
from jax.experimental.pallas import tpu_sc as plsc

_H = 768
_E = 64
_N = 32768
_NW = 32
_L = 16
_NC = 1
_CT = _N // _NC
_TPW = _CT // _NW
_MB = 4
_NS = 4
_TS = _TPW // _NS


def _mm_body(w_ref, x_ref, o_ref):
    for j in range(_MB):
        r = lax.dot_general(
            w_ref[...], x_ref[pl.ds(j * _TPW, _TPW), :],
            dimension_numbers=(((1,), (1,)), ((), ())),
            preferred_element_type=jnp.float32,
        )
        for s in range(_NS):
            o_ref[j, s] = r[:, s * _TS:(s + 1) * _TS]


def _matmul_logits_t(gw, x_chunk):
    return pl.pallas_call(
        _mm_body,
        grid=(_NW // _MB,),
        in_specs=[
            pl.BlockSpec((_E, _H), lambda i: (0, 0)),
            pl.BlockSpec((_MB * _TPW, _H), lambda i: (i, 0)),
        ],
        out_specs=pl.BlockSpec((_MB, _NS, _E, _TS), lambda i: (i, 0, 0, 0)),
        out_shape=jax.ShapeDtypeStruct((_NW, _NS, _E, _TS), jnp.float32),
    )(gw, x_chunk)


def _topk_sc_body(l_hbm, g1_hbm, g2_hbm, i1_hbm, i2_hbm,
                  blk, g1v, g2v, i1v, i2v, *sems):
    wid = lax.axis_index("s") * 2 + lax.axis_index("c")
    copies = [
        pltpu.async_copy(l_hbm.at[wid, s], blk.at[s], sems[s])
        for s in range(_NS)
    ]

    for s in range(_NS):
        copies[s].wait()

        def group(g, _, s=s):
            t0 = s * _TS + g * _L
            m1 = blk[s, 0, pl.ds(g * _L, _L)]
            i1 = jnp.zeros((_L,), jnp.int32)
            m2 = jnp.full((_L,), -jnp.inf, jnp.float32)
            i2 = jnp.zeros((_L,), jnp.int32)
            for e in range(1, _E):
                v = blk[s, e, pl.ds(g * _L, _L)]
                ev = jnp.full((_L,), e, jnp.int32)
                gt1 = v > m1
                gt2 = v > m2
                m2 = jnp.where(gt1, m1, jnp.where(gt2, v, m2))
                i2 = jnp.where(gt1, i1, jnp.where(gt2, ev, i2))
                m1 = jnp.where(gt1, v, m1)
                i1 = jnp.where(gt1, ev, i1)
            ed = jnp.exp(m2 - m1)
            g1 = 1.0 / (1.0 + ed)
            g1v[pl.ds(t0, _L)] = g1
            g2v[pl.ds(t0, _L)] = 1.0 - g1
            i1v[pl.ds(t0, _L)] = i1
            i2v[pl.ds(t0, _L)] = i2
            return ()

        lax.fori_loop(0, _TS // _L, group, ())

    base = wid * _TPW
    outs = [
        pltpu.async_copy(g1v, g1_hbm.at[pl.ds(base, _TPW)], sems[0]),
        pltpu.async_copy(g2v, g2_hbm.at[pl.ds(base, _TPW)], sems[1]),
        pltpu.async_copy(i1v, i1_hbm.at[pl.ds(base, _TPW)], sems[2]),
        pltpu.async_copy(i2v, i2_hbm.at[pl.ds(base, _TPW)], sems[3]),
    ]
    for o in outs:
        o.wait()


def _topk_sc(logits_t):
    mesh = plsc.VectorSubcoreMesh(core_axis_name="c", subcore_axis_name="s")
    f = functools.partial(
        pl.kernel,
        mesh=mesh,
        out_type=[
            jax.ShapeDtypeStruct((_CT,), jnp.float32),
            jax.ShapeDtypeStruct((_CT,), jnp.float32),
            jax.ShapeDtypeStruct((_CT,), jnp.int32),
            jax.ShapeDtypeStruct((_CT,), jnp.int32),
        ],
        scratch_types=[
            pltpu.VMEM((_NS, _E, _TS), jnp.float32),
            pltpu.VMEM((_TPW,), jnp.float32),
            pltpu.VMEM((_TPW,), jnp.float32),
            pltpu.VMEM((_TPW,), jnp.int32),
            pltpu.VMEM((_TPW,), jnp.int32),
        ] + [pltpu.SemaphoreType.DMA] * _NS,
    )(_topk_sc_body)
    return f(logits_t)


def kernel(x, gate_weight):
    parts = []
    for c in range(_NC):
        logits_t = _matmul_logits_t(gate_weight, x[c * _CT:(c + 1) * _CT])
        parts.append(_topk_sc(logits_t))
    g1 = jnp.concatenate([p[0] for p in parts])
    g2 = jnp.concatenate([p[1] for p in parts])
    i1 = jnp.concatenate([p[2] for p in parts])
    i2 = jnp.concatenate([p[3] for p in parts])
    gates = jnp.stack([g1, g2], axis=-1)
    idx = jnp.stack([i1, i2], axis=-1)
    return (gates, idx)


# speedup vs baseline: 1.0017x; 1.0017x over previous
import functools

import jax
import jax.numpy as jnp
from jax import lax
from jax.experimental import pallas as pl
from jax.experimental.pallas import tpu as pltpu
from jax.experimental.pallas import tpu_sc as plsc

_H = 768
_E = 64
_N = 32768
_NW = 32
_L = 16
_NC = 1
_CT = _N // _NC
_TPW = _CT // _NW
_MB = 4
_NS = 4
_TS = _TPW // _NS


def _mm_body(w_ref, x_ref, o_ref):
    for j in range(_MB):
        r = lax.dot_general(
            w_ref[...], x_ref[pl.ds(j * _TPW, _TPW), :],
            dimension_numbers=(((1,), (1,)), ((), ())),
            preferred_element_type=jnp.float32,
        )
        for s in range(_NS):
            o_ref[j, s] = r[:, s * _TS:(s + 1) * _TS]


def _matmul_logits_t(gw, x_chunk):
    return pl.pallas_call(
        _mm_body,
        grid=(_NW // _MB,),
        in_specs=[
            pl.BlockSpec((_E, _H), lambda i: (0, 0)),
            pl.BlockSpec((_MB * _TPW, _H), lambda i: (i, 0)),
        ],
        out_specs=pl.BlockSpec((_MB, _NS, _E, _TS), lambda i: (i, 0, 0, 0)),
        out_shape=jax.ShapeDtypeStruct((_NW, _NS, _E, _TS), jnp.float32),
    )(gw, x_chunk)


def _topk_sc_body(l_hbm, g1_hbm, g2_hbm, i1_hbm, i2_hbm,
                  blk, g1v, g2v, i1v, i2v, *sems):
    wid = lax.axis_index("s") * 2 + lax.axis_index("c")
    pltpu.sync_copy(l_hbm.at[wid], blk)

    for s in range(_NS):

        def group(g, _, s=s):
            t0 = s * _TS + g * _L
            m1 = blk[s, 0, pl.ds(g * _L, _L)]
            i1 = jnp.zeros((_L,), jnp.int32)
            m2 = jnp.full((_L,), -jnp.inf, jnp.float32)
            i2 = jnp.zeros((_L,), jnp.int32)
            for e in range(1, _E):
                v = blk[s, e, pl.ds(g * _L, _L)]
                ev = jnp.full((_L,), e, jnp.int32)
                gt1 = v > m1
                gt2 = v > m2
                m2 = jnp.where(gt1, m1, jnp.where(gt2, v, m2))
                i2 = jnp.where(gt1, i1, jnp.where(gt2, ev, i2))
                m1 = jnp.where(gt1, v, m1)
                i1 = jnp.where(gt1, ev, i1)
            ed = jnp.exp(m2 - m1)
            g1 = 1.0 / (1.0 + ed)
            g1v[pl.ds(t0, _L)] = g1
            g2v[pl.ds(t0, _L)] = 1.0 - g1
            i1v[pl.ds(t0, _L)] = i1
            i2v[pl.ds(t0, _L)] = i2
            return ()

        lax.fori_loop(0, _TS // _L, group, ())

    base = wid * _TPW
    outs = [
        pltpu.async_copy(g1v, g1_hbm.at[pl.ds(base, _TPW)], sems[0]),
        pltpu.async_copy(g2v, g2_hbm.at[pl.ds(base, _TPW)], sems[1]),
        pltpu.async_copy(i1v, i1_hbm.at[pl.ds(base, _TPW)], sems[2]),
        pltpu.async_copy(i2v, i2_hbm.at[pl.ds(base, _TPW)], sems[3]),
    ]
    for o in outs:
        o.wait()


def _topk_sc(logits_t):
    mesh = plsc.VectorSubcoreMesh(core_axis_name="c", subcore_axis_name="s")
    f = functools.partial(
        pl.kernel,
        mesh=mesh,
        out_type=[
            jax.ShapeDtypeStruct((_CT,), jnp.float32),
            jax.ShapeDtypeStruct((_CT,), jnp.float32),
            jax.ShapeDtypeStruct((_CT,), jnp.int32),
            jax.ShapeDtypeStruct((_CT,), jnp.int32),
        ],
        scratch_types=[
            pltpu.VMEM((_NS, _E, _TS), jnp.float32),
            pltpu.VMEM((_TPW,), jnp.float32),
            pltpu.VMEM((_TPW,), jnp.float32),
            pltpu.VMEM((_TPW,), jnp.int32),
            pltpu.VMEM((_TPW,), jnp.int32),
        ] + [pltpu.SemaphoreType.DMA] * _NS,
    )(_topk_sc_body)
    return f(logits_t)


def kernel(x, gate_weight):
    parts = []
    for c in range(_NC):
        logits_t = _matmul_logits_t(gate_weight, x[c * _CT:(c + 1) * _CT])
        parts.append(_topk_sc(logits_t))
    g1 = jnp.concatenate([p[0] for p in parts])
    g2 = jnp.concatenate([p[1] for p in parts])
    i1 = jnp.concatenate([p[2] for p in parts])
    i2 = jnp.concatenate([p[3] for p in parts])
    gates = jnp.stack([g1, g2], axis=-1)
    idx = jnp.stack([i1, i2], axis=-1)
    return (gates, idx)


# speedup vs baseline: 1.0030x; 1.0013x over previous
import functools

import jax
import jax.numpy as jnp
from jax import lax
from jax.experimental import pallas as pl
from jax.experimental.pallas import tpu as pltpu
from jax.experimental.pallas import tpu_sc as plsc

_H = 768
_E = 64
_N = 32768
_NW = 32
_L = 16
_NC = 1
_CT = _N // _NC
_TPW = _CT // _NW
_MB = 4
_NS = 4
_TS = _TPW // _NS


def _mm_body(w_ref, x_ref, o_ref):
    for j in range(_MB):
        o_ref[j] = lax.dot_general(
            w_ref[...], x_ref[pl.ds(j * _TPW, _TPW), :],
            dimension_numbers=(((1,), (1,)), ((), ())),
            preferred_element_type=jnp.float32,
        )


def _matmul_logits_t(gw, x_chunk):
    return pl.pallas_call(
        _mm_body,
        grid=(_NW // _MB,),
        in_specs=[
            pl.BlockSpec((_E, _H), lambda i: (0, 0)),
            pl.BlockSpec((_MB * _TPW, _H), lambda i: (i, 0)),
        ],
        out_specs=pl.BlockSpec((_MB, _E, _TPW), lambda i: (i, 0, 0)),
        out_shape=jax.ShapeDtypeStruct((_NW, _E, _TPW), jnp.float32),
    )(gw, x_chunk)


def _topk_sc_body(l_hbm, g1_hbm, g2_hbm, i1_hbm, i2_hbm,
                  blk, g1v, g2v, i1v, i2v, *sems):
    wid = lax.axis_index("s") * 2 + lax.axis_index("c")
    copies = [
        pltpu.async_copy(l_hbm.at[wid, :, pl.ds(s * _TS, _TS)],
                         blk.at[s], sems[s])
        for s in range(_NS)
    ]

    for s in range(_NS):
        copies[s].wait()

        def group(g, _, s=s):
            t0 = s * _TS + g * _L
            m1 = blk[s, 0, pl.ds(g * _L, _L)]
            i1 = jnp.zeros((_L,), jnp.int32)
            m2 = jnp.full((_L,), -jnp.inf, jnp.float32)
            i2 = jnp.zeros((_L,), jnp.int32)
            for e in range(1, _E):
                v = blk[s, e, pl.ds(g * _L, _L)]
                ev = jnp.full((_L,), e, jnp.int32)
                gt1 = v > m1
                gt2 = v > m2
                m2 = jnp.where(gt1, m1, jnp.where(gt2, v, m2))
                i2 = jnp.where(gt1, i1, jnp.where(gt2, ev, i2))
                m1 = jnp.where(gt1, v, m1)
                i1 = jnp.where(gt1, ev, i1)
            ed = jnp.exp(m2 - m1)
            g1 = 1.0 / (1.0 + ed)
            g1v[pl.ds(t0, _L)] = g1
            g2v[pl.ds(t0, _L)] = 1.0 - g1
            i1v[pl.ds(t0, _L)] = i1
            i2v[pl.ds(t0, _L)] = i2
            return ()

        lax.fori_loop(0, _TS // _L, group, ())

    base = wid * _TPW
    outs = [
        pltpu.async_copy(g1v, g1_hbm.at[pl.ds(base, _TPW)], sems[0]),
        pltpu.async_copy(g2v, g2_hbm.at[pl.ds(base, _TPW)], sems[1]),
        pltpu.async_copy(i1v, i1_hbm.at[pl.ds(base, _TPW)], sems[2]),
        pltpu.async_copy(i2v, i2_hbm.at[pl.ds(base, _TPW)], sems[3]),
    ]
    for o in outs:
        o.wait()


def _topk_sc(logits_t):
    mesh = plsc.VectorSubcoreMesh(core_axis_name="c", subcore_axis_name="s")
    f = functools.partial(
        pl.kernel,
        mesh=mesh,
        out_type=[
            jax.ShapeDtypeStruct((_CT,), jnp.float32),
            jax.ShapeDtypeStruct((_CT,), jnp.float32),
            jax.ShapeDtypeStruct((_CT,), jnp.int32),
            jax.ShapeDtypeStruct((_CT,), jnp.int32),
        ],
        scratch_types=[
            pltpu.VMEM((_NS, _E, _TS), jnp.float32),
            pltpu.VMEM((_TPW,), jnp.float32),
            pltpu.VMEM((_TPW,), jnp.float32),
            pltpu.VMEM((_TPW,), jnp.int32),
            pltpu.VMEM((_TPW,), jnp.int32),
        ] + [pltpu.SemaphoreType.DMA] * _NS,
    )(_topk_sc_body)
    return f(logits_t)


def kernel(x, gate_weight):
    parts = []
    for c in range(_NC):
        logits_t = _matmul_logits_t(gate_weight, x[c * _CT:(c + 1) * _CT])
        parts.append(_topk_sc(logits_t))
    g1 = jnp.concatenate([p[0] for p in parts])
    g2 = jnp.concatenate([p[1] for p in parts])
    i1 = jnp.concatenate([p[2] for p in parts])
    i2 = jnp.concatenate([p[3] for p in parts])
    gates = jnp.stack([g1, g2], axis=-1)
    idx = jnp.stack([i1, i2], axis=-1)
    return (gates, idx)


# speedup vs baseline: 1.0799x; 1.0767x over previous
import functools

import jax
import jax.numpy as jnp
from jax import lax
from jax.experimental import pallas as pl
from jax.experimental.pallas import tpu as pltpu
from jax.experimental.pallas import tpu_sc as plsc

_H = 768
_E = 64
_N = 32768
_NW = 32
_L = 16
_NC = 1
_CT = _N // _NC
_TPW = _CT // _NW
_MB = 4
_NI = 4


def _mm_body(w_ref, x_ref, o_ref):
    for j in range(_MB):
        o_ref[j] = lax.dot_general(
            w_ref[...], x_ref[pl.ds(j * _TPW, _TPW), :],
            dimension_numbers=(((1,), (1,)), ((), ())),
            preferred_element_type=jnp.float32,
        )


def _matmul_logits_t(gw, x_chunk):
    return pl.pallas_call(
        _mm_body,
        grid=(_NW // _MB,),
        in_specs=[
            pl.BlockSpec((_E, _H), lambda i: (0, 0)),
            pl.BlockSpec((_MB * _TPW, _H), lambda i: (i, 0)),
        ],
        out_specs=pl.BlockSpec((_MB, _E, _TPW), lambda i: (i, 0, 0)),
        out_shape=jax.ShapeDtypeStruct((_NW, _E, _TPW), jnp.float32),
    )(gw, x_chunk)


def _topk_sc_body(l_hbm, g1_hbm, g2_hbm, i1_hbm, i2_hbm,
                  blk, g1v, g2v, i1v, i2v, *sems):
    wid = lax.axis_index("s") * 2 + lax.axis_index("c")
    pltpu.sync_copy(l_hbm.at[wid], blk)

    def group(g, _):
        t0 = g * (_NI * _L)
        m1 = [blk[0, pl.ds(t0 + c * _L, _L)] for c in range(_NI)]
        i1 = [jnp.zeros((_L,), jnp.int32) for _ in range(_NI)]
        m2 = [jnp.full((_L,), -jnp.inf, jnp.float32) for _ in range(_NI)]
        i2 = [jnp.zeros((_L,), jnp.int32) for _ in range(_NI)]
        for e in range(1, _E):
            ev = jnp.full((_L,), e, jnp.int32)
            for c in range(_NI):
                v = blk[e, pl.ds(t0 + c * _L, _L)]
                gt1 = v > m1[c]
                gt2 = v > m2[c]
                m2[c] = jnp.where(gt1, m1[c], jnp.where(gt2, v, m2[c]))
                i2[c] = jnp.where(gt1, i1[c], jnp.where(gt2, ev, i2[c]))
                m1[c] = jnp.where(gt1, v, m1[c])
                i1[c] = jnp.where(gt1, ev, i1[c])
        for c in range(_NI):
            ed = jnp.exp(m2[c] - m1[c])
            g1 = 1.0 / (1.0 + ed)
            tc = t0 + c * _L
            g1v[pl.ds(tc, _L)] = g1
            g2v[pl.ds(tc, _L)] = 1.0 - g1
            i1v[pl.ds(tc, _L)] = i1[c]
            i2v[pl.ds(tc, _L)] = i2[c]
        return ()

    lax.fori_loop(0, _TPW // (_NI * _L), group, ())

    base = wid * _TPW
    outs = [
        pltpu.async_copy(g1v, g1_hbm.at[pl.ds(base, _TPW)], sems[0]),
        pltpu.async_copy(g2v, g2_hbm.at[pl.ds(base, _TPW)], sems[1]),
        pltpu.async_copy(i1v, i1_hbm.at[pl.ds(base, _TPW)], sems[2]),
        pltpu.async_copy(i2v, i2_hbm.at[pl.ds(base, _TPW)], sems[3]),
    ]
    for o in outs:
        o.wait()


def _topk_sc(logits_t):
    mesh = plsc.VectorSubcoreMesh(core_axis_name="c", subcore_axis_name="s")
    f = functools.partial(
        pl.kernel,
        mesh=mesh,
        out_type=[
            jax.ShapeDtypeStruct((_CT,), jnp.float32),
            jax.ShapeDtypeStruct((_CT,), jnp.float32),
            jax.ShapeDtypeStruct((_CT,), jnp.int32),
            jax.ShapeDtypeStruct((_CT,), jnp.int32),
        ],
        scratch_types=[
            pltpu.VMEM((_E, _TPW), jnp.float32),
            pltpu.VMEM((_TPW,), jnp.float32),
            pltpu.VMEM((_TPW,), jnp.float32),
            pltpu.VMEM((_TPW,), jnp.int32),
            pltpu.VMEM((_TPW,), jnp.int32),
        ] + [pltpu.SemaphoreType.DMA] * 4,
    )(_topk_sc_body)
    return f(logits_t)


def kernel(x, gate_weight):
    parts = []
    for c in range(_NC):
        logits_t = _matmul_logits_t(gate_weight, x[c * _CT:(c + 1) * _CT])
        parts.append(_topk_sc(logits_t))
    g1 = jnp.concatenate([p[0] for p in parts])
    g2 = jnp.concatenate([p[1] for p in parts])
    i1 = jnp.concatenate([p[2] for p in parts])
    i2 = jnp.concatenate([p[3] for p in parts])
    gates = jnp.stack([g1, g2], axis=-1)
    idx = jnp.stack([i1, i2], axis=-1)
    return (gates, idx)
